# baseline (speedup 1.0000x reference)
import math

import jax
import jax.numpy as jnp
from jax import lax
from jax.experimental import pallas as pl
from jax.experimental.pallas import tpu as pltpu
from jax.experimental.pallas import tpu_sc as plsc

D_MODEL = 768
INV_SQRT_D = 1.0 / math.sqrt(D_MODEL)
LANES = 16
CHUNK = 64


def _embed_call(n_tokens):
    info = plsc.get_sparse_core_info()
    nc, ns = info.num_cores, info.num_subcores
    nw = nc * ns
    tpw = n_tokens // nw
    assert n_tokens % (nw * CHUNK) == 0
    n_chunks = tpw // CHUNK
    mesh = plsc.VectorSubcoreMesh(core_axis_name="c", subcore_axis_name="s")

    def body(ids_hbm, cids_hbm, word_hbm, task_hbm, seg_hbm, out_hbm,
             idx_v, cid_v, task_v, seg_v, comb_v, rows0, rows1,
             gsem0, gsem1, osem0, osem1):
        wid = lax.axis_index("s") * nc + lax.axis_index("c")
        base = wid * tpw

        pltpu.sync_copy(ids_hbm.at[pl.ds(base, tpw)], idx_v)
        pltpu.sync_copy(cids_hbm.at[pl.ds(base, tpw)], cid_v)

        def gather(c, buf, gsem):
            return pltpu.async_copy(
                word_hbm.at[idx_v.at[pl.ds(c * CHUNK, CHUNK)]], buf, gsem)

        g = {}
        o = {}
        rows_b = [rows0, rows1]
        gsems = [gsem0, gsem1]
        osems = [osem0, osem1]
        g[0] = gather(0, rows0, gsem0)

        pltpu.sync_copy(task_hbm, task_v)
        pltpu.sync_copy(seg_hbm, seg_v)

        def build(j, carry):
            sl = pl.ds(j * LANES, LANES)
            for i in range(9):
                comb_v[pl.ds(i * D_MODEL + j * LANES, LANES)] = (
                    task_v[i // 3, sl] + seg_v[i % 3, sl]) * INV_SQRT_D
            return carry

        lax.fori_loop(0, D_MODEL // LANES, build, 0)

        iota = lax.iota(jnp.int32, LANES)

        def add_chunk(c, buf):
            def add_tok(t, carry):
                ci_bc = plsc.load_gather(
                    cid_v, [jnp.broadcast_to(c * CHUNK + t, (LANES,))])
                addr = ci_bc * D_MODEL + iota
                for j in range(D_MODEL // LANES):
                    vals = plsc.load_gather(comb_v, [addr])
                    plsc.addupdate(buf.at[t, pl.ds(j * LANES, LANES)], vals)
                    addr = addr + LANES
                return carry

            lax.fori_loop(0, CHUNK, add_tok, 0, unroll=4)

        for c in range(n_chunks):
            p = c % 2
            if c + 1 < n_chunks:
                if c >= 1:
                    o[c - 1].wait()
                g[c + 1] = gather(c + 1, rows_b[1 - p], gsems[1 - p])
            g[c].wait()
            add_chunk(c, rows_b[p])
            o[c] = pltpu.async_copy(
                rows_b[p], out_hbm.at[pl.ds(base + c * CHUNK, CHUNK)], osems[p])
        o[n_chunks - 2].wait()
        o[n_chunks - 1].wait()

    return pl.kernel(
        body,
        mesh=mesh,
        compiler_params=pltpu.CompilerParams(
            use_tc_tiling_on_sc=True, needs_layout_passes=False),
        out_type=jax.ShapeDtypeStruct((n_tokens, D_MODEL), jnp.float32),
        scratch_types=[
            pltpu.VMEM((2048 // 8,), jnp.int32),
            pltpu.VMEM((2048 // 8,), jnp.int32),
            pltpu.VMEM((3, D_MODEL), jnp.float32),
            pltpu.VMEM((3, D_MODEL), jnp.float32),
            pltpu.VMEM((9 * D_MODEL,), jnp.float32),
            pltpu.VMEM((CHUNK, D_MODEL), jnp.float32),
            pltpu.VMEM((CHUNK, D_MODEL), jnp.float32),
            pltpu.SemaphoreType.DMA,
            pltpu.SemaphoreType.DMA,
            pltpu.SemaphoreType.DMA,
            pltpu.SemaphoreType.DMA,
        ],
    )


def kernel(input_ids, task_ids, segment_ids, word_table, task_table, segment_table):
    b, l = input_ids.shape
    n = b * l
    ids = input_ids.reshape(n).astype(jnp.int32)
    cids = (task_ids.reshape(n) * 3 + segment_ids.reshape(n)).astype(jnp.int32)
    call = _embed_call(n)
    out = call(ids, cids, word_table, task_table, segment_table)
    return out.reshape(b, l, D_MODEL)


# speedup vs baseline: 1.0716x; 1.0716x over previous
import math

import jax
import jax.numpy as jnp
from jax import lax
from jax.experimental import pallas as pl
from jax.experimental.pallas import tpu as pltpu
from jax.experimental.pallas import tpu_sc as plsc

D_MODEL = 768
INV_SQRT_D = 1.0 / math.sqrt(D_MODEL)
LANES = 16
CHUNK = 32
NBUF = 3


def _embed_call(n_tokens):
    info = plsc.get_sparse_core_info()
    nc, ns = info.num_cores, info.num_subcores
    nw = nc * ns
    tpw = n_tokens // nw
    assert n_tokens % (nw * CHUNK) == 0
    n_chunks = tpw // CHUNK
    mesh = plsc.VectorSubcoreMesh(core_axis_name="c", subcore_axis_name="s")

    def body(ids_hbm, cids_hbm, word_hbm, task_hbm, seg_hbm, out_hbm,
             idx_v, cid_v, task_v, seg_v, comb_v, rows0, rows1, rows2,
             gsem0, gsem1, gsem2, osem0, osem1, osem2):
        wid = lax.axis_index("s") * nc + lax.axis_index("c")
        base = wid * tpw

        pltpu.sync_copy(ids_hbm.at[pl.ds(base, tpw)], idx_v)
        pltpu.sync_copy(cids_hbm.at[pl.ds(base, tpw)], cid_v)

        def gather(c, buf, gsem):
            return pltpu.async_copy(
                word_hbm.at[idx_v.at[pl.ds(c * CHUNK, CHUNK)]], buf, gsem)

        g = {}
        o = {}
        rows_b = [rows0, rows1, rows2]
        gsems = [gsem0, gsem1, gsem2]
        osems = [osem0, osem1, osem2]
        g[0] = gather(0, rows0, gsem0)

        pltpu.sync_copy(task_hbm, task_v)
        pltpu.sync_copy(seg_hbm, seg_v)

        def build(j, carry):
            sl = pl.ds(j * LANES, LANES)
            for i in range(9):
                comb_v[pl.ds(i * D_MODEL + j * LANES, LANES)] = (
                    task_v[i // 3, sl] + seg_v[i % 3, sl]) * INV_SQRT_D
            return carry

        lax.fori_loop(0, D_MODEL // LANES, build, 0)

        iota = lax.iota(jnp.int32, LANES)

        def add_chunk(c, buf):
            def add_tok(t, carry):
                ci_bc = plsc.load_gather(
                    cid_v, [jnp.broadcast_to(c * CHUNK + t, (LANES,))])
                addr = ci_bc * D_MODEL + iota
                for j in range(D_MODEL // LANES):
                    vals = plsc.load_gather(comb_v, [addr])
                    plsc.addupdate(buf.at[t, pl.ds(j * LANES, LANES)], vals)
                    addr = addr + LANES
                return carry

            lax.fori_loop(0, CHUNK, add_tok, 0, unroll=2)

        for c in range(n_chunks):
            p = c % NBUF
            if c + 1 < n_chunks:
                pn = (c + 1) % NBUF
                if c + 1 >= NBUF:
                    o[c + 1 - NBUF].wait()
                g[c + 1] = gather(c + 1, rows_b[pn], gsems[pn])
            g[c].wait()
            add_chunk(c, rows_b[p])
            o[c] = pltpu.async_copy(
                rows_b[p], out_hbm.at[pl.ds(base + c * CHUNK, CHUNK)], osems[p])
        for c in range(max(0, n_chunks - NBUF), n_chunks):
            o[c].wait()

    return pl.kernel(
        body,
        mesh=mesh,
        compiler_params=pltpu.CompilerParams(
            use_tc_tiling_on_sc=True, needs_layout_passes=False),
        out_type=jax.ShapeDtypeStruct((n_tokens, D_MODEL), jnp.float32),
        scratch_types=[
            pltpu.VMEM((2048 // 8,), jnp.int32),
            pltpu.VMEM((2048 // 8,), jnp.int32),
            pltpu.VMEM((3, D_MODEL), jnp.float32),
            pltpu.VMEM((3, D_MODEL), jnp.float32),
            pltpu.VMEM((9 * D_MODEL,), jnp.float32),
            pltpu.VMEM((CHUNK, D_MODEL), jnp.float32),
            pltpu.VMEM((CHUNK, D_MODEL), jnp.float32),
            pltpu.VMEM((CHUNK, D_MODEL), jnp.float32),
            pltpu.SemaphoreType.DMA,
            pltpu.SemaphoreType.DMA,
            pltpu.SemaphoreType.DMA,
            pltpu.SemaphoreType.DMA,
            pltpu.SemaphoreType.DMA,
            pltpu.SemaphoreType.DMA,
        ],
    )


def kernel(input_ids, task_ids, segment_ids, word_table, task_table, segment_table):
    b, l = input_ids.shape
    n = b * l
    ids = input_ids.reshape(n).astype(jnp.int32)
    cids = (task_ids.reshape(n) * 3 + segment_ids.reshape(n)).astype(jnp.int32)
    call = _embed_call(n)
    out = call(ids, cids, word_table, task_table, segment_table)
    return out.reshape(b, l, D_MODEL)


# speedup vs baseline: 1.2521x; 1.1684x over previous
import math

import jax
import jax.numpy as jnp
from jax import lax
from jax.experimental import pallas as pl
from jax.experimental.pallas import tpu as pltpu
from jax.experimental.pallas import tpu_sc as plsc

D_MODEL = 768
INV_SQRT_D = 1.0 / math.sqrt(D_MODEL)
LANES = 16
CHUNK = 32
NBUF = 3


def _embed_call(n_tokens):
    info = plsc.get_sparse_core_info()
    nc, ns = info.num_cores, info.num_subcores
    nw = nc * ns
    tpw = n_tokens // nw
    assert n_tokens % (nw * CHUNK) == 0
    n_chunks = tpw // CHUNK
    mesh = plsc.VectorSubcoreMesh(core_axis_name="c", subcore_axis_name="s")

    def body(ids_hbm, cids_hbm, word_hbm, task_hbm, seg_hbm, out_hbm,
             idx_v, cid_v, task_v, seg_v, comb_v, rows0, rows1, rows2,
             gsem0, gsem1, gsem2, osem0, osem1, osem2):
        wid = lax.axis_index("s") * nc + lax.axis_index("c")
        base = wid * tpw

        pltpu.sync_copy(ids_hbm.at[pl.ds(base, tpw)], idx_v)
        pltpu.sync_copy(cids_hbm.at[pl.ds(base, tpw)], cid_v)

        def gather(c, buf, gsem):
            return pltpu.async_copy(
                word_hbm.at[idx_v.at[pl.ds(c * CHUNK, CHUNK)]], buf, gsem)

        g = {}
        o = {}
        rows_b = [rows0, rows1, rows2]
        gsems = [gsem0, gsem1, gsem2]
        osems = [osem0, osem1, osem2]
        g[0] = gather(0, rows0, gsem0)

        pltpu.sync_copy(task_hbm, task_v)
        pltpu.sync_copy(seg_hbm, seg_v)

        def build(j, carry):
            sl = pl.ds(j * LANES, LANES)
            for i in range(9):
                comb_v[pl.ds(i * D_MODEL + j * LANES, LANES)] = (
                    task_v[i // 3, sl] + seg_v[i % 3, sl]) * INV_SQRT_D
            return carry

        lax.fori_loop(0, D_MODEL // LANES, build, 0)

        iota = lax.iota(jnp.int32, LANES)

        def add_chunk(c, buf):
            def add_tok(t, carry):
                ci_bc = plsc.load_gather(
                    cid_v, [jnp.broadcast_to(c * CHUNK + t, (LANES,))])
                addr = ci_bc * D_MODEL + iota
                for j in range(D_MODEL // LANES):
                    vals = plsc.load_gather(comb_v, [addr])
                    plsc.addupdate(buf.at[t, pl.ds(j * LANES, LANES)], vals)
                    addr = addr + LANES
                return carry

            @plsc.parallel_loop(0, CHUNK, step=1, unroll=2)
            def _add_loop(t):
                add_tok(t, 0)

        for c in range(n_chunks):
            p = c % NBUF
            if c + 1 < n_chunks:
                pn = (c + 1) % NBUF
                if c + 1 >= NBUF:
                    o[c + 1 - NBUF].wait()
                g[c + 1] = gather(c + 1, rows_b[pn], gsems[pn])
            g[c].wait()
            add_chunk(c, rows_b[p])
            o[c] = pltpu.async_copy(
                rows_b[p], out_hbm.at[pl.ds(base + c * CHUNK, CHUNK)], osems[p])
        for c in range(max(0, n_chunks - NBUF), n_chunks):
            o[c].wait()

    return pl.kernel(
        body,
        mesh=mesh,
        compiler_params=pltpu.CompilerParams(
            use_tc_tiling_on_sc=True, needs_layout_passes=False),
        out_type=jax.ShapeDtypeStruct((n_tokens, D_MODEL), jnp.float32),
        scratch_types=[
            pltpu.VMEM((2048 // 8,), jnp.int32),
            pltpu.VMEM((2048 // 8,), jnp.int32),
            pltpu.VMEM((3, D_MODEL), jnp.float32),
            pltpu.VMEM((3, D_MODEL), jnp.float32),
            pltpu.VMEM((9 * D_MODEL,), jnp.float32),
            pltpu.VMEM((CHUNK, D_MODEL), jnp.float32),
            pltpu.VMEM((CHUNK, D_MODEL), jnp.float32),
            pltpu.VMEM((CHUNK, D_MODEL), jnp.float32),
            pltpu.SemaphoreType.DMA,
            pltpu.SemaphoreType.DMA,
            pltpu.SemaphoreType.DMA,
            pltpu.SemaphoreType.DMA,
            pltpu.SemaphoreType.DMA,
            pltpu.SemaphoreType.DMA,
        ],
    )


def kernel(input_ids, task_ids, segment_ids, word_table, task_table, segment_table):
    b, l = input_ids.shape
    n = b * l
    ids = input_ids.reshape(n).astype(jnp.int32)
    cids = (task_ids.reshape(n) * 3 + segment_ids.reshape(n)).astype(jnp.int32)
    call = _embed_call(n)
    out = call(ids, cids, word_table, task_table, segment_table)
    return out.reshape(b, l, D_MODEL)


# speedup vs baseline: 1.4367x; 1.1474x over previous
import math

import jax
import jax.numpy as jnp
from jax import lax
from jax.experimental import pallas as pl
from jax.experimental.pallas import tpu as pltpu
from jax.experimental.pallas import tpu_sc as plsc

D_MODEL = 768
INV_SQRT_D = 1.0 / math.sqrt(D_MODEL)
LANES = 16
CHUNK = 32
NBUF = 3


def _embed_call(n_tokens):
    info = plsc.get_sparse_core_info()
    nc, ns = info.num_cores, info.num_subcores
    nw = nc * ns
    tpw = n_tokens // nw
    assert n_tokens % (nw * CHUNK) == 0
    n_chunks = tpw // CHUNK
    mesh = plsc.VectorSubcoreMesh(core_axis_name="c", subcore_axis_name="s")

    def body(ids_hbm, cids_hbm, word_hbm, task_hbm, seg_hbm, out_hbm,
             idx_v, cid_v, task_v, seg_v, comb_v, rows0, rows1, rows2,
             gsem0, gsem1, gsem2, osem0, osem1, osem2):
        wid = lax.axis_index("s") * nc + lax.axis_index("c")
        base = wid * tpw

        pltpu.sync_copy(ids_hbm.at[pl.ds(base, tpw)], idx_v)
        pltpu.sync_copy(cids_hbm.at[pl.ds(base, tpw)], cid_v)

        def gather(c, buf, gsem):
            return pltpu.async_copy(
                word_hbm.at[idx_v.at[pl.ds(c * CHUNK, CHUNK)]], buf, gsem)

        g = {}
        o = {}
        rows_b = [rows0, rows1, rows2]
        gsems = [gsem0, gsem1, gsem2]
        osems = [osem0, osem1, osem2]
        g[0] = gather(0, rows0, gsem0)

        pltpu.sync_copy(task_hbm, task_v)
        pltpu.sync_copy(seg_hbm, seg_v)

        def build(j, carry):
            sl = pl.ds(j * LANES, LANES)
            for i in range(9):
                comb_v[pl.ds(i * D_MODEL + j * LANES, LANES)] = (
                    task_v[i // 3, sl] + seg_v[i % 3, sl]) * INV_SQRT_D
            return carry

        lax.fori_loop(0, D_MODEL // LANES, build, 0)

        iota = lax.iota(jnp.int32, LANES)

        def add_chunk(c, buf):
            def add_tok(t, carry):
                ci_bc = plsc.load_gather(
                    cid_v, [jnp.broadcast_to(c * CHUNK + t, (LANES,))])
                addr = ci_bc * D_MODEL + iota
                for j in range(D_MODEL // LANES):
                    vals = plsc.load_gather(comb_v, [addr])
                    plsc.addupdate(buf.at[t, pl.ds(j * LANES, LANES)], vals)
                    addr = addr + LANES
                return carry

            @plsc.parallel_loop(0, CHUNK, step=1, unroll=4)
            def _add_loop(t):
                add_tok(t, 0)

        for c in range(n_chunks):
            p = c % NBUF
            if c + 1 < n_chunks:
                pn = (c + 1) % NBUF
                if c + 1 >= NBUF:
                    o[c + 1 - NBUF].wait()
                g[c + 1] = gather(c + 1, rows_b[pn], gsems[pn])
            g[c].wait()
            add_chunk(c, rows_b[p])
            o[c] = pltpu.async_copy(
                rows_b[p], out_hbm.at[pl.ds(base + c * CHUNK, CHUNK)], osems[p])
        for c in range(max(0, n_chunks - NBUF), n_chunks):
            o[c].wait()

    return pl.kernel(
        body,
        mesh=mesh,
        compiler_params=pltpu.CompilerParams(
            use_tc_tiling_on_sc=True, needs_layout_passes=False),
        out_type=jax.ShapeDtypeStruct((n_tokens, D_MODEL), jnp.float32),
        scratch_types=[
            pltpu.VMEM((2048 // 8,), jnp.int32),
            pltpu.VMEM((2048 // 8,), jnp.int32),
            pltpu.VMEM((3, D_MODEL), jnp.float32),
            pltpu.VMEM((3, D_MODEL), jnp.float32),
            pltpu.VMEM((9 * D_MODEL,), jnp.float32),
            pltpu.VMEM((CHUNK, D_MODEL), jnp.float32),
            pltpu.VMEM((CHUNK, D_MODEL), jnp.float32),
            pltpu.VMEM((CHUNK, D_MODEL), jnp.float32),
            pltpu.SemaphoreType.DMA,
            pltpu.SemaphoreType.DMA,
            pltpu.SemaphoreType.DMA,
            pltpu.SemaphoreType.DMA,
            pltpu.SemaphoreType.DMA,
            pltpu.SemaphoreType.DMA,
        ],
    )


def kernel(input_ids, task_ids, segment_ids, word_table, task_table, segment_table):
    b, l = input_ids.shape
    n = b * l
    ids = input_ids.reshape(n).astype(jnp.int32)
    cids = (task_ids.reshape(n) * 3 + segment_ids.reshape(n)).astype(jnp.int32)
    call = _embed_call(n)
    out = call(ids, cids, word_table, task_table, segment_table)
    return out.reshape(b, l, D_MODEL)
